# Initial kernel scaffold; baseline (speedup 1.0000x reference)
import jax
import jax.numpy as jnp
from jax.experimental import pallas as pl


def kernel(h, g, Wq, bq, Wk, bk, Wv, bv, Ws, bs, WO, bO, ln1_g, ln1_b, ln2_g, ln2_b, W1, b1f, W2, b2f):
    raise NotImplementedError("write your pallas kernel here")


# speedup vs baseline: 20.6442x; 20.6442x over previous
import functools
import math

import jax
import jax.numpy as jnp
from jax import lax
from jax.experimental import pallas as pl
from jax.experimental.pallas import tpu as pltpu
from jax.experimental.pallas import tpu_sc as plsc

N, E, D, H, C = 10000, 320000, 128, 8, 16

NC, NS, L = 2, 16, 16
NW = NC * NS
EPT = E // NW
B = 80
NCHUNK = EPT // B
ROW = 144
RPS = N // NS

RB = 1000


_GDN = lax.GatherDimensionNumbers(
    offset_dims=(), collapsed_slice_dims=(0,), start_index_map=(0,))


def _vtake(x, idx):
    return lax.gather(x, idx[:, None], _GDN, (1,),
                      mode=lax.GatherScatterMode.PROMISE_IN_BOUNDS)


def _elu(x):
    return jnp.where(x > 0, x, jnp.exp(jnp.minimum(x, 0.0)) - 1.0)


def _layer_norm(x, g, b, eps=1e-5):
    mu = jnp.mean(x, axis=-1, keepdims=True)
    var = jnp.mean((x - mu) ** 2, axis=-1, keepdims=True)
    return (x - mu) / jnp.sqrt(var + eps) * g + b


def _pre_body(h_ref, wq_ref, bq_ref, wk_ref, bk_ref, wv_ref, bv_ref,
              q_ref, k_ref, v_ref):
    hb = h_ref[...]
    q_ref[...] = (jnp.dot(hb, wq_ref[...], preferred_element_type=jnp.float32)
                  + bq_ref[...]) * (1.0 / math.sqrt(C))
    k_ref[...] = jnp.dot(hb, wk_ref[...], preferred_element_type=jnp.float32) + bk_ref[...]
    v_ref[...] = jnp.dot(hb, wv_ref[...], preferred_element_type=jnp.float32) + bv_ref[...]


def _pre(h, Wq, bq, Wk, bk, Wv, bv):
    full = pl.BlockSpec((D, D), lambda i: (0, 0))
    brow = pl.BlockSpec((1, D), lambda i: (0, 0))
    blk = pl.BlockSpec((RB, D), lambda i: (i, 0))
    return pl.pallas_call(
        _pre_body,
        grid=(N // RB,),
        in_specs=[blk, full, brow, full, brow, full, brow],
        out_specs=[blk, blk, blk],
        out_shape=[jax.ShapeDtypeStruct((N, D), jnp.float32)] * 3,
    )(h, Wq, bq.reshape(1, D), Wk, bk.reshape(1, D), Wv, bv.reshape(1, D))


def _pass_a_body(q_hbm, k_hbm, dst_hbm, src_hbm, alpha_hbm, tmax_hbm,
                 dst_v, src_v, q_v, k_v, a_v, mx_v, sem_q, sem_k):
    wid = lax.axis_index("s") * NC + lax.axis_index("c")
    ebase = wid * EPT
    lane = lax.iota(jnp.int32, 16)
    lane15 = lane == 15
    neg = jnp.full((L,), -jnp.inf, jnp.float32)

    def chunk_body(ci, mx):
        e0 = ebase + ci * B
        pltpu.sync_copy(dst_hbm.at[pl.ds(e0, B)], dst_v)
        pltpu.sync_copy(src_hbm.at[pl.ds(e0, B)], src_v)
        cq = pltpu.async_copy(q_hbm.at[dst_v], q_v, sem_q)
        ck = pltpu.async_copy(k_hbm.at[src_v], k_v, sem_k)
        cq.wait()
        ck.wait()

        def edge_body(e, mxi):
            for h in range(H):
                qv = q_v[e, pl.ds(h * C, C)]
                kv = k_v[e, pl.ds(h * C, C)]
                cum = plsc.cumsum(qv * kv)
                plsc.store_scatter(
                    a_v, [jnp.full((L,), e * H + h, jnp.int32)], cum,
                    mask=lane15)
                mxi = jnp.maximum(mxi, jnp.where(lane15, cum, neg))
            return mxi

        mx = pl.loop(0, B, init_carry=mx)(edge_body)
        pltpu.sync_copy(a_v, alpha_hbm.at[pl.ds(e0 * H, B * H)])
        return mx

    mx = pl.loop(0, NCHUNK, init_carry=neg)(chunk_body)
    mx_v[...] = mx
    pltpu.sync_copy(mx_v, tmax_hbm.at[wid])


def _pass_a(q, k, dst, src):
    mesh = plsc.VectorSubcoreMesh(core_axis_name="c", subcore_axis_name="s")
    f = pl.kernel(
        _pass_a_body,
        out_type=(jax.ShapeDtypeStruct((E * H,), jnp.float32),
                  jax.ShapeDtypeStruct((NW, L), jnp.float32)),
        mesh=mesh,
        compiler_params=pltpu.CompilerParams(needs_layout_passes=False, use_tc_tiling_on_sc=False),
        scratch_types=[
            pltpu.VMEM((B,), jnp.int32),
            pltpu.VMEM((B,), jnp.int32),
            pltpu.VMEM((B, D), jnp.float32),
            pltpu.VMEM((B, D), jnp.float32),
            pltpu.VMEM((B * H,), jnp.float32),
            pltpu.VMEM((L,), jnp.float32),
            pltpu.SemaphoreType.DMA,
            pltpu.SemaphoreType.DMA,
        ],
    )
    return f(q, k, dst, src)


def _pass_b_body(alpha_hbm, dst_hbm, src_hbm, v_hbm, m_hbm, agg_hbm,
                 dst_v, src_v, v_v, o_v, a_v, m_v, agg_sh, sem_v):
    cid = lax.axis_index("c")
    sid = lax.axis_index("s")
    wid = sid * NC + cid
    ebase = wid * EPT
    lane = lax.iota(jnp.int32, 16)
    rowoff = jnp.where(lane < 8, 0, 1)
    coloff = D + (lane & 7)

    pltpu.sync_copy(m_hbm, m_v)
    mreg = m_v[...]

    def zo_body(r):
        for cgroup in range(ROW // L):
            o_v[r, pl.ds(cgroup * L, L)] = jnp.zeros((L,), jnp.float32)
    pl.loop(0, B)(zo_body)

    for z in range(7):
        pltpu.sync_copy(o_v, agg_sh.at[pl.ds(sid * RPS + z * B, B)])
    pltpu.sync_copy(o_v.at[pl.ds(0, RPS - 7 * B)],
                    agg_sh.at[pl.ds(sid * RPS + 7 * B, RPS - 7 * B)])
    plsc.subcore_barrier()

    def chunk_body(ci):
        e0 = ebase + ci * B
        pltpu.sync_copy(dst_hbm.at[pl.ds(e0, B)], dst_v)
        pltpu.sync_copy(src_hbm.at[pl.ds(e0, B)], src_v)
        pltpu.sync_copy(alpha_hbm.at[pl.ds(e0 * H, B * H)], a_v)
        pltpu.async_copy(v_hbm.at[src_v], v_v, sem_v).wait()

        def pair_body(i):
            a = a_v[pl.ds(i * L, L)]
            ex = jnp.exp(a - mreg)
            r0 = 2 * i
            plsc.store_scatter(o_v, [r0 + rowoff, coloff], ex)
            for h in range(H):
                bc0 = _vtake(ex, jnp.full((L,), h, jnp.int32))
                bc1 = _vtake(ex, jnp.full((L,), H + h, jnp.int32))
                o_v[r0, pl.ds(h * C, C)] = v_v[r0, pl.ds(h * C, C)] * bc0
                o_v[r0 + 1, pl.ds(h * C, C)] = v_v[r0 + 1, pl.ds(h * C, C)] * bc1
        pl.loop(0, B // 2)(pair_body)
        pltpu.sync_copy(o_v, agg_sh.at[dst_v], add=True)

    pl.loop(0, NCHUNK)(chunk_body)
    plsc.subcore_barrier()
    pltpu.sync_copy(agg_sh.at[pl.ds(sid * RPS, RPS)],
                    agg_hbm.at[cid, pl.ds(sid * RPS, RPS)])


def _pass_b(alpha, dst, src, v, mvec):
    mesh = plsc.VectorSubcoreMesh(core_axis_name="c", subcore_axis_name="s")
    f = pl.kernel(
        _pass_b_body,
        out_type=jax.ShapeDtypeStruct((NC, N, ROW), jnp.float32),
        mesh=mesh,
        compiler_params=pltpu.CompilerParams(needs_layout_passes=False, use_tc_tiling_on_sc=False),
        scratch_types=[
            pltpu.VMEM((B,), jnp.int32),
            pltpu.VMEM((B,), jnp.int32),
            pltpu.VMEM((B, D), jnp.float32),
            pltpu.VMEM((B, ROW), jnp.float32),
            pltpu.VMEM((B * H,), jnp.float32),
            pltpu.VMEM((L,), jnp.float32),
            pltpu.VMEM_SHARED((N, ROW), jnp.float32),
            pltpu.SemaphoreType.DMA,
        ],
    )
    return f(alpha, dst, src, v, mvec)


def _post_body(wv0_ref, wv1_ref, d0_ref, d1_ref, h_ref,
               ws_ref, bs_ref, wo_ref, bo_ref,
               ln1g_ref, ln1b_ref, ln2g_ref, ln2b_ref,
               w1_ref, b1_ref, w2_ref, b2_ref, out_ref):
    hb = h_ref[...]
    agg = (wv0_ref[...] + wv1_ref[...]) / (d0_ref[...] + d1_ref[...] + 1e-16)
    a = agg + jnp.dot(hb, ws_ref[...], preferred_element_type=jnp.float32) + bs_ref[...]
    at = jnp.dot(_elu(a), wo_ref[...], preferred_element_type=jnp.float32) + bo_ref[...]
    h1 = _layer_norm(at + hb, ln1g_ref[...], ln1b_ref[...])
    f1 = _elu(jnp.dot(h1, w1_ref[...], preferred_element_type=jnp.float32) + b1_ref[...])
    ffn = jnp.dot(f1, w2_ref[...], preferred_element_type=jnp.float32) + b2_ref[...]
    out_ref[...] = _layer_norm(ffn + h1, ln2g_ref[...], ln2b_ref[...])


def _post(wv0, wv1, d0, d1, h, Ws, bs, WO, bO, ln1_g, ln1_b, ln2_g, ln2_b,
          W1, b1f, W2, b2f):
    blk = pl.BlockSpec((RB, D), lambda i: (i, 0))
    full = pl.BlockSpec((D, D), lambda i: (0, 0))
    brow = pl.BlockSpec((1, D), lambda i: (0, 0))
    w1s = pl.BlockSpec((D, 2 * D), lambda i: (0, 0))
    b1s = pl.BlockSpec((1, 2 * D), lambda i: (0, 0))
    w2s = pl.BlockSpec((2 * D, D), lambda i: (0, 0))
    return pl.pallas_call(
        _post_body,
        grid=(N // RB,),
        in_specs=[blk, blk, blk, blk, blk,
                  full, brow, full, brow,
                  brow, brow, brow, brow,
                  w1s, b1s, w2s, brow],
        out_specs=blk,
        out_shape=jax.ShapeDtypeStruct((N, D), jnp.float32),
    )(wv0, wv1, d0, d1, h,
      Ws, bs.reshape(1, D), WO, bO.reshape(1, D),
      ln1_g.reshape(1, D), ln1_b.reshape(1, D),
      ln2_g.reshape(1, D), ln2_b.reshape(1, D),
      W1, b1f.reshape(1, 2 * D), W2, b2f.reshape(1, D))


def kernel(h, g, Wq, bq, Wk, bk, Wv, bv, Ws, bs, WO, bO,
           ln1_g, ln1_b, ln2_g, ln2_b, W1, b1f, W2, b2f):
    src = g[0].astype(jnp.int32)
    dst = g[1].astype(jnp.int32)

    q, k, v = _pre(h, Wq, bq, Wk, bk, Wv, bv)
    alpha, tmax = _pass_a(q, k, dst, src)
    m8 = jnp.max(tmax.reshape(NW * 2, H), axis=0)
    mvec = jnp.concatenate([m8, m8])
    agg2 = _pass_b(alpha, dst, src, v, mvec)

    wv0 = agg2[0, :, :D]
    wv1 = agg2[1, :, :D]
    d0 = agg2[0, :, D:D + H]
    d1 = agg2[1, :, D:D + H]
    d0e = jnp.broadcast_to(d0[:, :, None], (N, H, C)).reshape(N, D)
    d1e = jnp.broadcast_to(d1[:, :, None], (N, H, C)).reshape(N, D)

    return _post(wv0, wv1, d0e, d1e, h, Ws, bs, WO, bO,
                 ln1_g, ln1_b, ln2_g, ln2_b, W1, b1f, W2, b2f)
